# baseline (speedup 1.0000x reference)
import functools
import jax
import jax.numpy as jnp
from jax import lax
from jax.experimental import pallas as pl
from jax.experimental.pallas import tpu as pltpu, tpu_sc as plsc

H = 224
W = 224
HO = H // 2
WO = W // 2
NV = W // 16
NB = 256
NBV = NB // 16
IMGS = 768
IMGS_SC = 608
IMGS_PER_WORKER = IMGS_SC // 32

_MESH = plsc.VectorSubcoreMesh(core_axis_name="c", subcore_axis_name="s")


def _sortable_i32(xf):
    xb = plsc.bitcast(xf, jnp.int32)
    flip = (xb >> 31) | jnp.int32(-(2**31))
    return xb ^ flip


def _digit(w_i, shift):
    d = (plsc.bitcast(w_i, jnp.uint32) >> jnp.uint32(shift)) & jnp.uint32(NB - 1)
    return plsc.bitcast(d, jnp.int32)


NROWS = 14


def _body(x_hbm, out_hbm, ximg, oimg, *rest):
    srow_list = list(rest[: NROWS // 2])
    sets = rest[NROWS // 2 :]
    wid = lax.axis_index("s") * 2 + lax.axis_index("c")
    iota = lax.iota(jnp.int32, 16)
    ones = jnp.ones((16,), jnp.int32)
    zeros = jnp.zeros((16,), jnp.int32)

    ctxs = [tuple(sets[4 * r : 4 * r + 4]) for r in range(NROWS)]
    srows = [srow_list[r // 2] for r in range(NROWS)]

    def sort_group(h0):
        def hist_zero():
            for i in range(NBV):
                for (_, _, hist, _) in ctxs:
                    hist[pl.ds(16 * i, 16)] = zeros

        def scan_bases():
            carries = [jnp.int32(0)] * NROWS
            for i in range(NBV):
                hvs = [c[2][pl.ds(16 * i, 16)] for c in ctxs]
                cs = [plsc.cumsum(hv) for hv in hvs]
                for r, (_, _, _, base) in enumerate(ctxs):
                    base[pl.ds(16 * i, 16)] = cs[r] - hvs[r] + carries[r]
                carries = [carries[r] + cs[r][15] for r in range(NROWS)]

        hist_zero()
        for v in range(NV):
            keys = [
                _sortable_i32(ximg[h0 + r, pl.ds(16 * v, 16)])
                for r in range(NROWS)
            ]
            for r, (_, _, hist, _) in enumerate(ctxs):
                plsc.addupdate_scatter(hist, [_digit(keys[r], 0)], ones)
        scan_bases()
        hist_zero()
        for v in range(NV):
            lane = iota + jnp.int32(16 * v)
            keys = [
                _sortable_i32(ximg[h0 + r, pl.ds(16 * v, 16)])
                for r in range(NROWS)
            ]
            ds_ = [_digit(k, 0) for k in keys]
            ws = [(k & jnp.int32(-256)) | lane for k in keys]
            rcs = [plsc.scan_count(d) for d in ds_]
            poss = [
                plsc.load_gather(ctxs[r][3], [ds_[r]]) + rcs[r][0] - 1
                for r in range(NROWS)
            ]
            for r in range(NROWS):
                plsc.store_scatter(ctxs[r][1], [poss[r]], ws[r])
            for r in range(NROWS):
                plsc.addupdate_scatter(
                    ctxs[r][3], [ds_[r]], rcs[r][0], mask=rcs[r][1])
            for r in range(NROWS):
                plsc.addupdate_scatter(ctxs[r][2], [_digit(ws[r], 8)], ones)
        for p, (si, di) in enumerate([(1, 0), (0, 1)], start=1):
            shift = 8 * p
            scan_bases()
            hist_zero()
            for v in range(NV):
                ws = [ctxs[r][si][pl.ds(16 * v, 16)] for r in range(NROWS)]
                gs = [_digit(w, shift) for w in ws]
                rcs = [plsc.scan_count(g) for g in gs]
                poss = [
                    plsc.load_gather(ctxs[r][3], [gs[r]]) + rcs[r][0] - 1
                    for r in range(NROWS)
                ]
                for r in range(NROWS):
                    plsc.store_scatter(ctxs[r][di], [poss[r]], ws[r])
                for r in range(NROWS):
                    plsc.addupdate_scatter(
                        ctxs[r][3], [gs[r]], rcs[r][0], mask=rcs[r][1])
                for r in range(NROWS):
                    plsc.addupdate_scatter(
                        ctxs[r][2], [_digit(ws[r], shift + 8)], ones)
        scan_bases()
        for v in range(NV):
            ws = [ctxs[r][1][pl.ds(16 * v, 16)] for r in range(NROWS)]
            gs = [_digit(w, 24) for w in ws]
            rcs = [plsc.scan_count(g) for g in gs]
            poss = [
                plsc.load_gather(ctxs[r][3], [gs[r]]) + rcs[r][0] - 1
                for r in range(NROWS)
            ]
            for r in range(NROWS):
                plsc.addupdate_scatter(
                    srows[r], [poss[r] >> 1], ws[r] & jnp.int32(255))
            for r in range(NROWS):
                plsc.addupdate_scatter(
                    ctxs[r][3], [gs[r]], rcs[r][0], mask=rcs[r][1])

    def img_body(jj, _):
        img = wid * IMGS_PER_WORKER + jj
        pltpu.sync_copy(x_hbm.at[pl.ds(img * H, H)], ximg)

        def group_body(g, _):
            for m in range(WO // 16):
                for sr in srow_list:
                    sr[pl.ds(16 * m, 16)] = zeros
            sort_group(NROWS * g)
            for m in range(WO // 16):
                hp = (NROWS // 2) * g
                for q, sr in enumerate(srow_list):
                    oimg[pl.ds((hp + q) * WO + 16 * m, 16)] = (
                        sr[pl.ds(16 * m, 16)].astype(jnp.float32) * 0.25)
            return 0

        lax.fori_loop(0, H // NROWS, group_body, 0)
        pltpu.sync_copy(oimg, out_hbm.at[img])
        return 0

    lax.fori_loop(0, IMGS_PER_WORKER, img_body, 0)


@functools.partial(
    pl.kernel,
    out_type=jax.ShapeDtypeStruct((IMGS_SC, HO * WO), jnp.float32),
    mesh=_MESH,
    compiler_params=pltpu.CompilerParams(needs_layout_passes=False),
    scratch_types=[
        pltpu.VMEM((H, W), jnp.float32),
        pltpu.VMEM((HO * WO,), jnp.float32),
    ] + (14 // 2) * [
        pltpu.VMEM((WO,), jnp.int32),
    ] + 14 * [
        pltpu.VMEM((W,), jnp.int32),
        pltpu.VMEM((W,), jnp.int32),
        pltpu.VMEM((NB,), jnp.int32),
        pltpu.VMEM((NB,), jnp.int32),
    ],
)
def _sc_kernel(x_hbm, out_hbm, *scratch):
    _body(x_hbm, out_hbm, *scratch)


def _tc_body(xt_ref, out_ref, s_ref):
    s_ref[...] = jnp.zeros((WO, H), jnp.float32)
    k_iota = jax.lax.broadcasted_iota(jnp.int32, (W, 1), 0)
    w_iota = jax.lax.broadcasted_iota(jnp.int32, (WO, 1), 0)

    def step(i, _):
        xt = xt_ref[0]
        rowi = xt_ref[0, pl.ds(i, 1), :]
        fi = i.astype(jnp.float32)
        lt = (xt < rowi).astype(jnp.float32)
        tie = jnp.where(k_iota < i, (xt == rowi).astype(jnp.float32), 0.0)
        r = jnp.sum(lt + tie, axis=0, keepdims=True)
        b = r.astype(jnp.int32) // 2
        s_ref[...] += fi * (b == w_iota).astype(jnp.float32)
        return 0

    jax.lax.fori_loop(0, W, step, 0)
    h_pair = jnp.where(
        jax.lax.broadcasted_iota(jnp.int32, (H, HO), 0) // 2
        == jax.lax.broadcasted_iota(jnp.int32, (H, HO), 1),
        jnp.float32(1.0),
        jnp.float32(0.0),
    )
    out_ref[0] = jnp.dot(s_ref[...], h_pair,
                         preferred_element_type=jnp.float32) * 0.25


def _tc_kernel(xt):
    n = xt.shape[0]
    return pl.pallas_call(
        _tc_body,
        grid=(n,),
        in_specs=[pl.BlockSpec((1, W, H), lambda i: (i, 0, 0))],
        out_specs=pl.BlockSpec((1, WO, HO), lambda i: (i, 0, 0)),
        out_shape=jax.ShapeDtypeStruct((n, WO, HO), jnp.float32),
        scratch_shapes=[pltpu.VMEM((WO, H), jnp.float32)],
    )(xt)


@jax.jit
def kernel(x):
    b, c, h, w = x.shape
    xi = x.reshape(b * c, h, w)
    x_sc = xi[:IMGS_SC].reshape(IMGS_SC * H, W)
    xt_tc = jnp.swapaxes(xi[IMGS_SC:], 1, 2)
    out_sc = _sc_kernel(x_sc).reshape(IMGS_SC, HO, WO)
    out_tc = jnp.swapaxes(_tc_kernel(xt_tc), 1, 2)
    out = jnp.concatenate([out_sc, out_tc], axis=0)
    return out.reshape(b, c, HO, WO)


# speedup vs baseline: 1.0387x; 1.0387x over previous
import functools
import jax
import jax.numpy as jnp
from jax import lax
from jax.experimental import pallas as pl
from jax.experimental.pallas import tpu as pltpu, tpu_sc as plsc

H = 224
W = 224
HO = H // 2
WO = W // 2
NV = W // 16
NB = 256
NBV = NB // 16
IMGS = 768
IMGS_SC = 672
IMGS_PER_WORKER = IMGS_SC // 32

_MESH = plsc.VectorSubcoreMesh(core_axis_name="c", subcore_axis_name="s")


def _sortable_i32(xf):
    xb = plsc.bitcast(xf, jnp.int32)
    flip = (xb >> 31) | jnp.int32(-(2**31))
    return xb ^ flip


def _digit(w_i, shift):
    d = (plsc.bitcast(w_i, jnp.uint32) >> jnp.uint32(shift)) & jnp.uint32(NB - 1)
    return plsc.bitcast(d, jnp.int32)


NROWS = 14


def _body(x_hbm, out_hbm, ximg, oimg, *rest):
    srow_list = list(rest[: NROWS // 2])
    sets = rest[NROWS // 2 :]
    wid = lax.axis_index("s") * 2 + lax.axis_index("c")
    iota = lax.iota(jnp.int32, 16)
    ones = jnp.ones((16,), jnp.int32)
    zeros = jnp.zeros((16,), jnp.int32)

    ctxs = [tuple(sets[4 * r : 4 * r + 4]) for r in range(NROWS)]
    srows = [srow_list[r // 2] for r in range(NROWS)]

    def sort_group(h0):
        def hist_zero():
            for i in range(NBV):
                for (_, _, hist, _) in ctxs:
                    hist[pl.ds(16 * i, 16)] = zeros

        def scan_bases():
            carries = [jnp.int32(0)] * NROWS
            for i in range(NBV):
                hvs = [c[2][pl.ds(16 * i, 16)] for c in ctxs]
                cs = [plsc.cumsum(hv) for hv in hvs]
                for r, (_, _, _, base) in enumerate(ctxs):
                    base[pl.ds(16 * i, 16)] = cs[r] - hvs[r] + carries[r]
                carries = [carries[r] + cs[r][15] for r in range(NROWS)]

        hist_zero()
        for v in range(NV):
            keys = [
                _sortable_i32(ximg[h0 + r, pl.ds(16 * v, 16)])
                for r in range(NROWS)
            ]
            for r, (_, _, hist, _) in enumerate(ctxs):
                plsc.addupdate_scatter(hist, [_digit(keys[r], 0)], ones)
        scan_bases()
        hist_zero()
        for v in range(NV):
            lane = iota + jnp.int32(16 * v)
            keys = [
                _sortable_i32(ximg[h0 + r, pl.ds(16 * v, 16)])
                for r in range(NROWS)
            ]
            ds_ = [_digit(k, 0) for k in keys]
            ws = [(k & jnp.int32(-256)) | lane for k in keys]
            rcs = [plsc.scan_count(d) for d in ds_]
            poss = [
                plsc.load_gather(ctxs[r][3], [ds_[r]]) + rcs[r][0] - 1
                for r in range(NROWS)
            ]
            for r in range(NROWS):
                plsc.store_scatter(ctxs[r][1], [poss[r]], ws[r])
            for r in range(NROWS):
                plsc.addupdate_scatter(
                    ctxs[r][3], [ds_[r]], rcs[r][0], mask=rcs[r][1])
            for r in range(NROWS):
                plsc.addupdate_scatter(ctxs[r][2], [_digit(ws[r], 8)], ones)
        for p, (si, di) in enumerate([(1, 0), (0, 1)], start=1):
            shift = 8 * p
            scan_bases()
            hist_zero()
            for v in range(NV):
                ws = [ctxs[r][si][pl.ds(16 * v, 16)] for r in range(NROWS)]
                gs = [_digit(w, shift) for w in ws]
                rcs = [plsc.scan_count(g) for g in gs]
                poss = [
                    plsc.load_gather(ctxs[r][3], [gs[r]]) + rcs[r][0] - 1
                    for r in range(NROWS)
                ]
                for r in range(NROWS):
                    plsc.store_scatter(ctxs[r][di], [poss[r]], ws[r])
                for r in range(NROWS):
                    plsc.addupdate_scatter(
                        ctxs[r][3], [gs[r]], rcs[r][0], mask=rcs[r][1])
                for r in range(NROWS):
                    plsc.addupdate_scatter(
                        ctxs[r][2], [_digit(ws[r], shift + 8)], ones)
        scan_bases()
        for v in range(NV):
            ws = [ctxs[r][1][pl.ds(16 * v, 16)] for r in range(NROWS)]
            gs = [_digit(w, 24) for w in ws]
            rcs = [plsc.scan_count(g) for g in gs]
            poss = [
                plsc.load_gather(ctxs[r][3], [gs[r]]) + rcs[r][0] - 1
                for r in range(NROWS)
            ]
            for r in range(NROWS):
                plsc.addupdate_scatter(
                    srows[r], [poss[r] >> 1], ws[r] & jnp.int32(255))
            for r in range(NROWS):
                plsc.addupdate_scatter(
                    ctxs[r][3], [gs[r]], rcs[r][0], mask=rcs[r][1])

    def img_body(jj, _):
        img = wid * IMGS_PER_WORKER + jj
        pltpu.sync_copy(x_hbm.at[pl.ds(img * H, H)], ximg)

        def group_body(g, _):
            for m in range(WO // 16):
                for sr in srow_list:
                    sr[pl.ds(16 * m, 16)] = zeros
            sort_group(NROWS * g)
            for m in range(WO // 16):
                hp = (NROWS // 2) * g
                for q, sr in enumerate(srow_list):
                    oimg[pl.ds((hp + q) * WO + 16 * m, 16)] = (
                        sr[pl.ds(16 * m, 16)].astype(jnp.float32) * 0.25)
            return 0

        lax.fori_loop(0, H // NROWS, group_body, 0)
        pltpu.sync_copy(oimg, out_hbm.at[img])
        return 0

    lax.fori_loop(0, IMGS_PER_WORKER, img_body, 0)


@functools.partial(
    pl.kernel,
    out_type=jax.ShapeDtypeStruct((IMGS_SC, HO * WO), jnp.float32),
    mesh=_MESH,
    compiler_params=pltpu.CompilerParams(needs_layout_passes=False),
    scratch_types=[
        pltpu.VMEM((H, W), jnp.float32),
        pltpu.VMEM((HO * WO,), jnp.float32),
    ] + (14 // 2) * [
        pltpu.VMEM((WO,), jnp.int32),
    ] + 14 * [
        pltpu.VMEM((W,), jnp.int32),
        pltpu.VMEM((W,), jnp.int32),
        pltpu.VMEM((NB,), jnp.int32),
        pltpu.VMEM((NB,), jnp.int32),
    ],
)
def _sc_kernel(x_hbm, out_hbm, *scratch):
    _body(x_hbm, out_hbm, *scratch)


def _tc_body(xt_ref, out_ref, s_ref):
    s_ref[...] = jnp.zeros((WO, H), jnp.float32)
    k_iota = jax.lax.broadcasted_iota(jnp.int32, (W, 1), 0)
    w_iota = jax.lax.broadcasted_iota(jnp.int32, (WO, 1), 0)

    def step(i, _):
        xt = xt_ref[0]
        rowi = xt_ref[0, pl.ds(i, 1), :]
        fi = i.astype(jnp.float32)
        lt = (xt < rowi).astype(jnp.float32)
        tie = jnp.where(k_iota < i, (xt == rowi).astype(jnp.float32), 0.0)
        r = jnp.sum(lt + tie, axis=0, keepdims=True)
        b = r.astype(jnp.int32) // 2
        s_ref[...] += fi * (b == w_iota).astype(jnp.float32)
        return 0

    jax.lax.fori_loop(0, W, step, 0)
    h_pair = jnp.where(
        jax.lax.broadcasted_iota(jnp.int32, (H, HO), 0) // 2
        == jax.lax.broadcasted_iota(jnp.int32, (H, HO), 1),
        jnp.float32(1.0),
        jnp.float32(0.0),
    )
    out_ref[0] = jnp.dot(s_ref[...], h_pair,
                         preferred_element_type=jnp.float32) * 0.25


def _tc_kernel(xt):
    n = xt.shape[0]
    return pl.pallas_call(
        _tc_body,
        grid=(n,),
        in_specs=[pl.BlockSpec((1, W, H), lambda i: (i, 0, 0))],
        out_specs=pl.BlockSpec((1, WO, HO), lambda i: (i, 0, 0)),
        out_shape=jax.ShapeDtypeStruct((n, WO, HO), jnp.float32),
        scratch_shapes=[pltpu.VMEM((WO, H), jnp.float32)],
    )(xt)


@jax.jit
def kernel(x):
    b, c, h, w = x.shape
    xi = x.reshape(b * c, h, w)
    x_sc = xi[:IMGS_SC].reshape(IMGS_SC * H, W)
    xt_tc = jnp.swapaxes(xi[IMGS_SC:], 1, 2)
    out_sc = _sc_kernel(x_sc).reshape(IMGS_SC, HO, WO)
    out_tc = jnp.swapaxes(_tc_kernel(xt_tc), 1, 2)
    out = jnp.concatenate([out_sc, out_tc], axis=0)
    return out.reshape(b, c, HO, WO)


# speedup vs baseline: 1.0547x; 1.0155x over previous
import functools
import jax
import jax.numpy as jnp
from jax import lax
from jax.experimental import pallas as pl
from jax.experimental.pallas import tpu as pltpu, tpu_sc as plsc

H = 224
W = 224
HO = H // 2
WO = W // 2
NV = W // 16
NB = 256
NBV = NB // 16
IMGS = 768
IMGS_PER_WORKER = IMGS // 32

_MESH = plsc.VectorSubcoreMesh(core_axis_name="c", subcore_axis_name="s")


def _sortable_i32(xf):
    xb = plsc.bitcast(xf, jnp.int32)
    flip = (xb >> 31) | jnp.int32(-(2**31))
    return xb ^ flip


def _digit(w_i, shift):
    d = (plsc.bitcast(w_i, jnp.uint32) >> jnp.uint32(shift)) & jnp.uint32(NB - 1)
    return plsc.bitcast(d, jnp.int32)


NROWS = 14


def _body(x_hbm, out_hbm, ximg, oimg, *rest):
    srow_list = list(rest[: NROWS // 2])
    sets = rest[NROWS // 2 :]
    wid = lax.axis_index("s") * 2 + lax.axis_index("c")
    iota = lax.iota(jnp.int32, 16)
    ones = jnp.ones((16,), jnp.int32)
    zeros = jnp.zeros((16,), jnp.int32)

    ctxs = [tuple(sets[4 * r : 4 * r + 4]) for r in range(NROWS)]
    srows = [srow_list[r // 2] for r in range(NROWS)]

    def sort_group(h0):
        def hist_zero():
            for i in range(NBV):
                for (_, _, hist, _) in ctxs:
                    hist[pl.ds(16 * i, 16)] = zeros

        def scan_bases():
            carries = [jnp.int32(0)] * NROWS
            for i in range(NBV):
                hvs = [c[2][pl.ds(16 * i, 16)] for c in ctxs]
                cs = [plsc.cumsum(hv) for hv in hvs]
                for r, (_, _, _, base) in enumerate(ctxs):
                    base[pl.ds(16 * i, 16)] = cs[r] - hvs[r] + carries[r]
                carries = [carries[r] + cs[r][15] for r in range(NROWS)]

        hist_zero()
        for v in range(NV):
            keys = [
                _sortable_i32(ximg[h0 + r, pl.ds(16 * v, 16)])
                for r in range(NROWS)
            ]
            for r, (_, _, hist, _) in enumerate(ctxs):
                plsc.addupdate_scatter(hist, [_digit(keys[r], 0)], ones)
        scan_bases()
        hist_zero()
        for v in range(NV):
            lane = iota + jnp.int32(16 * v)
            keys = [
                _sortable_i32(ximg[h0 + r, pl.ds(16 * v, 16)])
                for r in range(NROWS)
            ]
            ds_ = [_digit(k, 0) for k in keys]
            ws = [(k & jnp.int32(-256)) | lane for k in keys]
            rcs = [plsc.scan_count(d) for d in ds_]
            poss = [
                plsc.load_gather(ctxs[r][3], [ds_[r]]) + rcs[r][0] - 1
                for r in range(NROWS)
            ]
            for r in range(NROWS):
                plsc.store_scatter(ctxs[r][1], [poss[r]], ws[r])
            for r in range(NROWS):
                plsc.addupdate_scatter(
                    ctxs[r][3], [ds_[r]], rcs[r][0], mask=rcs[r][1])
            for r in range(NROWS):
                plsc.addupdate_scatter(ctxs[r][2], [_digit(ws[r], 8)], ones)
        for p, (si, di) in enumerate([(1, 0), (0, 1)], start=1):
            shift = 8 * p
            scan_bases()
            hist_zero()
            for v in range(NV):
                ws = [ctxs[r][si][pl.ds(16 * v, 16)] for r in range(NROWS)]
                gs = [_digit(w, shift) for w in ws]
                rcs = [plsc.scan_count(g) for g in gs]
                poss = [
                    plsc.load_gather(ctxs[r][3], [gs[r]]) + rcs[r][0] - 1
                    for r in range(NROWS)
                ]
                for r in range(NROWS):
                    plsc.store_scatter(ctxs[r][di], [poss[r]], ws[r])
                for r in range(NROWS):
                    plsc.addupdate_scatter(
                        ctxs[r][3], [gs[r]], rcs[r][0], mask=rcs[r][1])
                for r in range(NROWS):
                    plsc.addupdate_scatter(
                        ctxs[r][2], [_digit(ws[r], shift + 8)], ones)
        scan_bases()
        for v in range(NV):
            ws = [ctxs[r][1][pl.ds(16 * v, 16)] for r in range(NROWS)]
            gs = [_digit(w, 24) for w in ws]
            rcs = [plsc.scan_count(g) for g in gs]
            poss = [
                plsc.load_gather(ctxs[r][3], [gs[r]]) + rcs[r][0] - 1
                for r in range(NROWS)
            ]
            for r in range(NROWS):
                plsc.addupdate_scatter(
                    srows[r], [poss[r] >> 1], ws[r] & jnp.int32(255))
            for r in range(NROWS):
                plsc.addupdate_scatter(
                    ctxs[r][3], [gs[r]], rcs[r][0], mask=rcs[r][1])

    def img_body(jj, _):
        img = wid * IMGS_PER_WORKER + jj
        pltpu.sync_copy(x_hbm.at[pl.ds(img * H, H)], ximg)

        def group_body(g, _):
            for m in range(WO // 16):
                for sr in srow_list:
                    sr[pl.ds(16 * m, 16)] = zeros
            sort_group(NROWS * g)
            for m in range(WO // 16):
                hp = (NROWS // 2) * g
                for q, sr in enumerate(srow_list):
                    oimg[pl.ds((hp + q) * WO + 16 * m, 16)] = (
                        sr[pl.ds(16 * m, 16)].astype(jnp.float32) * 0.25)
            return 0

        lax.fori_loop(0, H // NROWS, group_body, 0)
        pltpu.sync_copy(oimg, out_hbm.at[img])
        return 0

    lax.fori_loop(0, IMGS_PER_WORKER, img_body, 0)


@functools.partial(
    pl.kernel,
    out_type=jax.ShapeDtypeStruct((IMGS, HO * WO), jnp.float32),
    mesh=_MESH,
    compiler_params=pltpu.CompilerParams(needs_layout_passes=False),
    scratch_types=[
        pltpu.VMEM((H, W), jnp.float32),
        pltpu.VMEM((HO * WO,), jnp.float32),
    ] + (14 // 2) * [
        pltpu.VMEM((WO,), jnp.int32),
    ] + 14 * [
        pltpu.VMEM((W,), jnp.int32),
        pltpu.VMEM((W,), jnp.int32),
        pltpu.VMEM((NB,), jnp.int32),
        pltpu.VMEM((NB,), jnp.int32),
    ],
)
def _sc_kernel(x_hbm, out_hbm, *scratch):
    _body(x_hbm, out_hbm, *scratch)


@jax.jit
def kernel(x):
    b, c, h, w = x.shape
    xf = x.reshape(b * c * h, w)
    out = _sc_kernel(xf)
    return out.reshape(b, c, HO, WO)
